# Initial kernel scaffold; baseline (speedup 1.0000x reference)
import jax
import jax.numpy as jnp
from jax.experimental import pallas as pl


def kernel(predictions, targets):
    raise NotImplementedError("write your pallas kernel here")


# speedup vs baseline: 4.6530x; 4.6530x over previous
import functools

import jax
import jax.numpy as jnp
from jax import lax
from jax.experimental import pallas as pl
from jax.experimental.pallas import tpu as pltpu
from jax.experimental.pallas import tpu_sc as plsc

_LANES = 16


def _build_kernel(B, W, C):
    info = plsc.get_sparse_core_info()
    NW = info.num_cores * info.num_subcores
    assert C == _LANES
    assert B % NW == 0
    rows_per_tile = B // NW
    CH = 8
    assert rows_per_tile % CH == 0
    n_chunks = rows_per_tile // CH
    chunk_words = CH * W * C
    pos_unroll = 8
    assert W % pos_unroll == 0
    n_steps = W // pos_unroll

    mesh = plsc.VectorSubcoreMesh(core_axis_name="c", subcore_axis_name="s")

    @functools.partial(
        pl.kernel,
        out_type=jax.ShapeDtypeStruct((NW * _LANES,), jnp.float32),
        mesh=mesh,
        compiler_params=pltpu.CompilerParams(needs_layout_passes=False),
        scratch_types=[
            pltpu.VMEM((chunk_words,), jnp.float32),
            pltpu.VMEM((chunk_words,), jnp.float32),
            pltpu.VMEM((rows_per_tile * W + _LANES,), jnp.int32),
            pltpu.VMEM((_LANES,), jnp.float32),
            pltpu.SemaphoreType.DMA,
            pltpu.SemaphoreType.DMA,
        ],
    )
    def run(pred_hbm, tgt_hbm, out_hbm, pbuf0, pbuf1, tbuf, obuf, sem0, sem1):
        cid = lax.axis_index("c")
        sid = lax.axis_index("s")
        wid = sid * info.num_cores + cid
        row_base = wid * rows_per_tile
        pred_base = row_base * (W * C)

        bufs = [pbuf0, pbuf1]
        sems = [sem0, sem1]
        handles = {}
        handles[0] = pltpu.async_copy(
            pred_hbm.at[pl.ds(pred_base, chunk_words)], pbuf0, sem0)
        if n_chunks > 1:
            handles[1] = pltpu.async_copy(
                pred_hbm.at[pl.ds(pred_base + chunk_words, chunk_words)],
                pbuf1, sem1)
        pltpu.sync_copy(tgt_hbm.at[pl.ds(row_base * W, rows_per_tile * W)],
                        tbuf.at[pl.ds(0, rows_per_tile * W)])

        iota = lax.iota(jnp.int32, _LANES)
        zeros_i = jnp.zeros((_LANES,), jnp.int32)
        obuf[...] = jnp.zeros((_LANES,), jnp.float32)

        for ci in range(n_chunks):
            handles[ci].wait()
            pbuf = bufs[ci % 2]

            def row_body(r, _, pbuf=pbuf, ci=ci):
                t_row = (ci * CH + r) * W

                def step_body(s, carry):
                    run_p, best_p, run_t, best_t = carry
                    p_off = (r * W + s * pos_unroll) * C
                    t_off = t_row + s * pos_unroll
                    tv = tbuf[pl.ds(t_off, _LANES)]
                    for j in range(pos_unroll):
                        x = pbuf[pl.ds(p_off + j * C, C)]
                        m = jnp.max(x)
                        f = plsc.all_reduce_ffs(x == m)
                        mask_p = iota == f
                        run_p = jnp.where(mask_p, run_p + 1, zeros_i)
                        best_p = jnp.maximum(best_p, run_p)
                        mask_t = iota == tv[j]
                        run_t = jnp.where(mask_t, run_t + 1, zeros_i)
                        best_t = jnp.maximum(best_t, run_t)
                    return run_p, best_p, run_t, best_t

                _, best_p, _, best_t = lax.fori_loop(
                    0, n_steps, step_body,
                    (zeros_i, zeros_i, zeros_i, zeros_i))
                d = (best_p - best_t).astype(jnp.float32)
                obuf[...] = obuf[...] + d * d
                return 0

            lax.fori_loop(0, CH, row_body, 0)

            nxt = ci + 2
            if nxt < n_chunks:
                handles[nxt] = pltpu.async_copy(
                    pred_hbm.at[pl.ds(pred_base + nxt * chunk_words,
                                      chunk_words)],
                    bufs[nxt % 2], sems[nxt % 2])

        pltpu.sync_copy(obuf, out_hbm.at[pl.ds(wid * _LANES, _LANES)])

    return run


def kernel(predictions, targets):
    B, W, C = predictions.shape
    run = _build_kernel(B, W, C)
    partials = run(predictions.reshape(-1),
                   targets.astype(jnp.int32).reshape(-1))
    return jnp.sum(partials) / jnp.float32(B * C)
